# baseline (speedup 1.0000x reference)
import functools

import jax
import jax.numpy as jnp
from jax import lax
from jax.experimental import pallas as pl
from jax.experimental.pallas import tpu as pltpu
from jax.experimental.pallas import tpu_sc as plsc

_USER_LEN = 1000000
_L_SIZE = 32
_BATCH = 16384

_NC = 2
_NS = 16
_NW = _NC * _NS
_B_PER_W = _BATCH // _NW
_CHUNK = 128
_NCHUNK = _B_PER_W // _CHUNK

_mesh = plsc.VectorSubcoreMesh(core_axis_name="c", subcore_axis_name="s")


@functools.partial(
    pl.kernel,
    mesh=_mesh,
    out_type=jax.ShapeDtypeStruct((_NW, _NCHUNK, _CHUNK, _L_SIZE), jnp.float32),
    scratch_types=[
        pltpu.VMEM((_NCHUNK, _CHUNK), jnp.int32),
        pltpu.VMEM((_NCHUNK, _CHUNK, _L_SIZE), jnp.float32),
        pltpu.SemaphoreType.DMA,
    ],
    compiler_params=pltpu.CompilerParams(use_tc_tiling_on_sc=False),
)
def _sc_gather(idx_hbm, table_hbm, out_hbm, idx_v, rows_v, sem):
    wid = lax.axis_index("s") * _NC + lax.axis_index("c")
    pltpu.sync_copy(idx_hbm.at[wid], idx_v)
    copies = [
        pltpu.async_copy(table_hbm.at[idx_v.at[c]], rows_v.at[c], sem)
        for c in range(_NCHUNK)
    ]
    for c in copies:
        c.wait()
    pltpu.sync_copy(rows_v, out_hbm.at[wid])


def kernel(users, uEmbed):
    idx = users.astype(jnp.int32).reshape(_NW, _NCHUNK, _CHUNK)
    out = _sc_gather(idx, uEmbed)
    return out.reshape(_BATCH, _L_SIZE)


# speedup vs baseline: 1.6510x; 1.6510x over previous
import functools

import jax
import jax.numpy as jnp
from jax import lax
from jax.experimental import pallas as pl
from jax.experimental.pallas import tpu as pltpu
from jax.experimental.pallas import tpu_sc as plsc

_USER_LEN = 1000000
_L_SIZE = 32
_BATCH = 16384

_NC = 2
_NS = 16
_NW = _NC * _NS
_B_PER_W = _BATCH // _NW

_mesh = plsc.VectorSubcoreMesh(core_axis_name="c", subcore_axis_name="s")


@functools.partial(
    pl.kernel,
    mesh=_mesh,
    out_type=jax.ShapeDtypeStruct((_NW, _B_PER_W, _L_SIZE), jnp.float32),
    scratch_types=[
        pltpu.VMEM((_B_PER_W,), jnp.int32),
        pltpu.VMEM((_B_PER_W, _L_SIZE), jnp.float32),
        pltpu.SemaphoreType.DMA,
    ],
)
def _sc_gather(idx_hbm, table_hbm, out_hbm, idx_v, rows_v, sem):
    wid = lax.axis_index("s") * _NC + lax.axis_index("c")
    pltpu.sync_copy(idx_hbm.at[wid], idx_v)

    def enq(g, _):
        v = idx_v[pl.ds(g * 16, 16)]
        base = g * 16
        for j in range(16):
            pltpu.async_copy(table_hbm.at[v[j]], rows_v.at[base + j], sem)
        return ()

    lax.fori_loop(0, _B_PER_W // 16, enq, ())
    pltpu.make_async_copy(
        table_hbm.at[pl.ds(0, _B_PER_W)], rows_v, sem
    ).wait()
    pltpu.sync_copy(rows_v, out_hbm.at[wid])


def kernel(users, uEmbed):
    idx = users.astype(jnp.int32).reshape(_NW, _B_PER_W)
    out = _sc_gather(idx, uEmbed)
    return out.reshape(_BATCH, _L_SIZE)


# speedup vs baseline: 4.6376x; 2.8090x over previous
import functools

import jax
import jax.numpy as jnp
from jax import lax
from jax.experimental import pallas as pl
from jax.experimental.pallas import tpu as pltpu
from jax.experimental.pallas import tpu_sc as plsc

_USER_LEN = 1000000
_L_SIZE = 32
_BATCH = 16384

_NC = 2
_NS = 16
_NW = _NC * _NS
_B_PER_W = _BATCH // _NW
_RING = 8

_mesh = plsc.VectorSubcoreMesh(core_axis_name="c", subcore_axis_name="s")


@functools.partial(
    pl.kernel,
    mesh=_mesh,
    out_type=jax.ShapeDtypeStruct((_L_SIZE, _BATCH), jnp.float32),
    scratch_types=[
        pltpu.VMEM((_B_PER_W + 16,), jnp.int32),
        pltpu.VMEM((_RING, _L_SIZE, 128), jnp.float32),
        pltpu.VMEM((_L_SIZE, _B_PER_W), jnp.float32),
        pltpu.SemaphoreType.DMA,
    ],
    compiler_params=pltpu.CompilerParams(
        disable_bounds_checks=True, needs_layout_passes=False
    ),
)
def _sc_gather(idx_hbm, tableT_hbm, outT_hbm, idx_v, blk, cols_v, sem):
    wid = lax.axis_index("s") * _NC + lax.axis_index("c")
    pltpu.sync_copy(idx_hbm.at[wid], idx_v.at[pl.ds(0, _B_PER_W)])

    c_lo = lax.iota(jnp.int32, 16)
    c_hi = c_lo + 16

    def row_r(i):
        return idx_v[pl.ds(i, 16)][0]

    def enqueue(i):
        r = row_r(i)
        base = pl.multiple_of((r >> 7) << 7, 128)
        pltpu.async_copy(
            tableT_hbm.at[:, pl.ds(base, 128)],
            blk.at[lax.rem(i, _RING)],
            sem,
        )

    def finish(i):
        pltpu.make_async_copy(
            tableT_hbm.at[:, pl.ds(0, 128)],
            blk.at[lax.rem(i, _RING)],
            sem,
        ).wait()
        r = row_r(i)
        m_vec = jnp.full((16,), r & 127, jnp.int32)
        s_vec = jnp.full((16,), lax.rem(i, _RING), jnp.int32)
        b_vec = jnp.full((16,), i, jnp.int32)
        v0 = plsc.load_gather(blk, [s_vec, c_lo, m_vec])
        v1 = plsc.load_gather(blk, [s_vec, c_hi, m_vec])
        plsc.store_scatter(cols_v, [c_lo, b_vec], v0)
        plsc.store_scatter(cols_v, [c_hi, b_vec], v1)

    def body(i, _):
        @pl.when(i >= _RING)
        def _():
            finish(i - _RING)

        enqueue(i)
        return ()

    lax.fori_loop(0, _B_PER_W, body, ())
    for k in range(_RING):
        finish(_B_PER_W - _RING + k)

    pltpu.sync_copy(
        cols_v,
        outT_hbm.at[:, pl.ds(pl.multiple_of(wid * _B_PER_W, 128), _B_PER_W)],
    )


def kernel(users, uEmbed):
    idx = users.astype(jnp.int32).reshape(_NW, _B_PER_W)
    outT = _sc_gather(idx, uEmbed.T)
    return outT.T


# speedup vs baseline: 4.7153x; 1.0167x over previous
import functools

import jax
import jax.numpy as jnp
from jax import lax
from jax.experimental import pallas as pl
from jax.experimental.pallas import tpu as pltpu
from jax.experimental.pallas import tpu_sc as plsc

_USER_LEN = 1000000
_L_SIZE = 32
_BATCH = 16384

_NC = 2
_NS = 16
_NW = _NC * _NS
_B_PER_W = _BATCH // _NW
_RING = 16

_mesh = plsc.VectorSubcoreMesh(core_axis_name="c", subcore_axis_name="s")


@functools.partial(
    pl.kernel,
    mesh=_mesh,
    out_type=jax.ShapeDtypeStruct((_L_SIZE, _BATCH), jnp.float32),
    scratch_types=[
        pltpu.VMEM((_B_PER_W + 16,), jnp.int32),
        pltpu.VMEM((_RING, _L_SIZE, 128), jnp.float32),
        pltpu.VMEM((_L_SIZE, _B_PER_W), jnp.float32),
        pltpu.SemaphoreType.DMA,
    ],
    compiler_params=pltpu.CompilerParams(
        disable_bounds_checks=True, needs_layout_passes=False
    ),
)
def _sc_gather(idx_hbm, tableT_hbm, outT_hbm, idx_v, blk, cols_v, sem):
    wid = lax.axis_index("s") * _NC + lax.axis_index("c")
    pltpu.sync_copy(idx_hbm.at[wid], idx_v.at[pl.ds(0, _B_PER_W)])

    c_lo = lax.iota(jnp.int32, 16)
    c_hi = c_lo + 16

    def row_r(i):
        return idx_v[pl.ds(i, 16)][0]

    def enqueue(i):
        r = row_r(i)
        base = pl.multiple_of((r >> 7) << 7, 128)
        pltpu.async_copy(
            tableT_hbm.at[:, pl.ds(base, 128)],
            blk.at[lax.rem(i, _RING)],
            sem,
        )

    def finish(i):
        pltpu.make_async_copy(
            tableT_hbm.at[:, pl.ds(0, 128)],
            blk.at[lax.rem(i, _RING)],
            sem,
        ).wait()
        r = row_r(i)
        m_vec = jnp.full((16,), r & 127, jnp.int32)
        s_vec = jnp.full((16,), lax.rem(i, _RING), jnp.int32)
        b_vec = jnp.full((16,), i, jnp.int32)
        v0 = plsc.load_gather(blk, [s_vec, c_lo, m_vec])
        v1 = plsc.load_gather(blk, [s_vec, c_hi, m_vec])
        plsc.store_scatter(cols_v, [c_lo, b_vec], v0)
        plsc.store_scatter(cols_v, [c_hi, b_vec], v1)

    def body(i, _):
        @pl.when(i >= _RING)
        def _():
            finish(i - _RING)

        enqueue(i)
        return ()

    lax.fori_loop(0, _B_PER_W, body, ())
    for k in range(_RING):
        finish(_B_PER_W - _RING + k)

    pltpu.sync_copy(
        cols_v,
        outT_hbm.at[:, pl.ds(pl.multiple_of(wid * _B_PER_W, 128), _B_PER_W)],
    )


def kernel(users, uEmbed):
    idx = users.astype(jnp.int32).reshape(_NW, _B_PER_W)
    outT = _sc_gather(idx, uEmbed.T)
    return outT.T
